# baseline (speedup 1.0000x reference)
import functools
import jax
import jax.numpy as jnp
from jax import lax
from jax.experimental import pallas as pl
from jax.experimental.pallas import tpu as pltpu
from jax.experimental.pallas import tpu_sc as plsc

B, N, D, E = 2, 2048, 768, 8
BN = B * N
T = 1024
NC, NS, L = 2, 16, 16
NW = NC * NS
TPW = BN // NW


def _gate_scores_kernel(x_ref, gate_ref, g_ref):
    g = jnp.dot(x_ref[...], gate_ref[...], preferred_element_type=jnp.float32)
    g_ref[...] = g.T


def _sc_route_kernel(g_hbm, w_hbm, g_v, w_v):
    wid = lax.axis_index("s") * NC + lax.axis_index("c")
    base = wid * TPW
    for e in range(E):
        pltpu.sync_copy(g_hbm.at[pl.ds(e * BN + base, TPW)],
                        g_v.at[pl.ds(e * TPW, TPW)])
    for j in range(TPW // L):
        p = [g_v[pl.ds(e * TPW + j * L, L)] for e in range(E)]
        m = p[0]
        for e in range(1, E):
            m = jnp.maximum(m, p[e])
        p = [jnp.exp(p[e] - m) for e in range(E)]
        m1 = p[0]
        for e in range(1, E):
            m1 = jnp.maximum(m1, p[e])
        i1 = jnp.full((L,), 0, jnp.int32)
        for e in range(E - 1, -1, -1):
            i1 = jnp.where(p[e] == m1, e, i1)
        p2 = [jnp.where(i1 == e, -1.0, p[e]) for e in range(E)]
        m2 = p2[0]
        for e in range(1, E):
            m2 = jnp.maximum(m2, p2[e])
        i2 = jnp.full((L,), 0, jnp.int32)
        for e in range(E - 1, -1, -1):
            i2 = jnp.where(p2[e] == m2, e, i2)
        ws = [jnp.where((i1 == e) | (i2 == e), p[e], 0.0) for e in range(E)]
        den = ws[0]
        for e in range(1, E):
            den = den + ws[e]
        inv = 1.0 / den
        for e in range(E):
            w_v[pl.ds(e * TPW + j * L, L)] = ws[e] * inv
    for e in range(E):
        pltpu.sync_copy(w_v.at[pl.ds(e * TPW, TPW)],
                        w_hbm.at[pl.ds(e * BN + base, TPW)])


_sc_route = functools.partial(
    pl.kernel,
    out_type=jax.ShapeDtypeStruct((E * BN,), jnp.float32),
    mesh=plsc.VectorSubcoreMesh(core_axis_name="c", subcore_axis_name="s"),
    scratch_types=[
        pltpu.VMEM((E * TPW,), jnp.float32),
        pltpu.VMEM((E * TPW,), jnp.float32),
    ],
)(_sc_route_kernel)


def _moe_block_kernel(x_ref, w_in_ref, w_ref, b_ref, o_ref):
    xb = x_ref[...]
    w = w_in_ref[...].T
    acc = jnp.dot(w, b_ref[...], preferred_element_type=jnp.float32)
    for e in range(E):
        y = jnp.dot(xb, w_ref[e], preferred_element_type=jnp.float32)
        acc = acc + w[:, e:e + 1] * y
    o_ref[...] = acc


def kernel(x, gate, W_e, b_e):
    xf = x.reshape(BN, D)
    g = pl.pallas_call(
        _gate_scores_kernel,
        grid=(BN // T,),
        in_specs=[
            pl.BlockSpec((T, D), lambda i: (i, 0)),
            pl.BlockSpec((D, E), lambda i: (0, 0)),
        ],
        out_specs=pl.BlockSpec((E, T), lambda i: (0, i)),
        out_shape=jax.ShapeDtypeStruct((E, BN), jnp.float32),
    )(xf, gate)
    w = _sc_route(g.reshape(E * BN)).reshape(E, BN)
    out = pl.pallas_call(
        _moe_block_kernel,
        grid=(BN // T,),
        in_specs=[
            pl.BlockSpec((T, D), lambda i: (i, 0)),
            pl.BlockSpec((E, T), lambda i: (0, i)),
            pl.BlockSpec((E, D, D), lambda i: (0, 0, 0)),
            pl.BlockSpec((E, D), lambda i: (0, 0)),
        ],
        out_specs=pl.BlockSpec((T, D), lambda i: (i, 0)),
        out_shape=jax.ShapeDtypeStruct((BN, D), jnp.float32),
        compiler_params=pltpu.CompilerParams(
            dimension_semantics=("arbitrary",),
        ),
    )(xf, w, W_e, b_e)
    return out.reshape(B, N, D)


# speedup vs baseline: 1.4967x; 1.4967x over previous
import jax
import jax.numpy as jnp
from jax.experimental import pallas as pl
from jax.experimental.pallas import tpu as pltpu

B, N, D, E = 2, 2048, 768, 8
TOPK = 2


def _moe_block_kernel(x_ref, gate_ref, w_ref, b_ref, o_ref):
    xb = x_ref[...]
    g = jnp.dot(xb, gate_ref[...], preferred_element_type=jnp.float32)
    g = g - jnp.max(g, axis=1, keepdims=True)
    p = jnp.exp(g)
    p = p / jnp.sum(p, axis=1, keepdims=True)
    col = jax.lax.broadcasted_iota(jnp.int32, p.shape, 1)
    m1 = jnp.max(p, axis=1, keepdims=True)
    i1 = jnp.min(jnp.where(p == m1, col, E), axis=1, keepdims=True)
    p2 = jnp.where(col == i1, -1.0, p)
    m2 = jnp.max(p2, axis=1, keepdims=True)
    i2 = jnp.min(jnp.where(p2 == m2, col, E), axis=1, keepdims=True)
    sel = (col == i1) | (col == i2)
    gsel = jnp.where(sel, p, 0.0)
    denom = jnp.maximum(jnp.sum(gsel, axis=1, keepdims=True), 1e-12)
    w = gsel / denom
    acc = jnp.dot(w, b_ref[...], preferred_element_type=jnp.float32)
    for e in range(E):
        y = jnp.dot(xb, w_ref[e], preferred_element_type=jnp.float32)
        acc = acc + w[:, e:e + 1] * y
    o_ref[...] = acc


def kernel(x, gate, W_e, b_e):
    BN = B * N
    T = 1024
    xf = x.reshape(BN, D)
    out = pl.pallas_call(
        _moe_block_kernel,
        grid=(BN // T,),
        in_specs=[
            pl.BlockSpec((T, D), lambda i: (i, 0)),
            pl.BlockSpec((D, E), lambda i: (0, 0)),
            pl.BlockSpec((E, D, D), lambda i: (0, 0, 0)),
            pl.BlockSpec((E, D), lambda i: (0, 0)),
        ],
        out_specs=pl.BlockSpec((T, D), lambda i: (i, 0)),
        out_shape=jax.ShapeDtypeStruct((BN, D), jnp.float32),
        compiler_params=pltpu.CompilerParams(
            dimension_semantics=("arbitrary",),
        ),
    )(xf, gate, W_e, b_e)
    return out.reshape(B, N, D)


# speedup vs baseline: 1.5027x; 1.0040x over previous
import jax
import jax.numpy as jnp
from jax.experimental import pallas as pl
from jax.experimental.pallas import tpu as pltpu

B, N, D, E = 2, 2048, 768, 8
TOPK = 2


def _moe_block_kernel(x_ref, gate_ref, w_ref, b_ref, o_ref):
    xb = x_ref[...]
    g = jnp.dot(xb, gate_ref[...], preferred_element_type=jnp.float32)
    g = g - jnp.max(g, axis=1, keepdims=True)
    p = jnp.exp(g)
    col = jax.lax.broadcasted_iota(jnp.int32, p.shape, 1)
    m1 = jnp.max(p, axis=1, keepdims=True)
    i1 = jnp.min(jnp.where(p == m1, col, E), axis=1, keepdims=True)
    p2 = jnp.where(col == i1, -1.0, p)
    m2 = jnp.max(p2, axis=1, keepdims=True)
    i2 = jnp.min(jnp.where(p2 == m2, col, E), axis=1, keepdims=True)
    sel = (col == i1) | (col == i2)
    gsel = jnp.where(sel, p, 0.0)
    denom = jnp.sum(gsel, axis=1, keepdims=True)
    w = gsel / denom
    acc = jnp.dot(w, b_ref[...], preferred_element_type=jnp.float32)
    for e in range(E):
        y = jnp.dot(xb, w_ref[e], preferred_element_type=jnp.float32)
        acc = acc + w[:, e:e + 1] * y
    o_ref[...] = acc


def kernel(x, gate, W_e, b_e):
    BN = B * N
    T = 1024
    xf = x.reshape(BN, D)
    out = pl.pallas_call(
        _moe_block_kernel,
        grid=(BN // T,),
        in_specs=[
            pl.BlockSpec((T, D), lambda i: (i, 0)),
            pl.BlockSpec((D, E), lambda i: (0, 0)),
            pl.BlockSpec((E, D, D), lambda i: (0, 0, 0)),
            pl.BlockSpec((E, D), lambda i: (0, 0)),
        ],
        out_specs=pl.BlockSpec((T, D), lambda i: (i, 0)),
        out_shape=jax.ShapeDtypeStruct((BN, D), jnp.float32),
        compiler_params=pltpu.CompilerParams(
            dimension_semantics=("arbitrary",),
        ),
    )(xf, gate, W_e, b_e)
    return out.reshape(B, N, D)
